# baseline (speedup 1.0000x reference)
import jax
import jax.numpy as jnp
from jax.experimental import pallas as pl
from jax.experimental.pallas import tpu as pltpu


def _gcn_kernel(tail, x_ref, w_ref, b_ref, adj_ref, adjt_ref, out_ref, s_ref):
    @pl.when(pl.program_id(0) == 0)
    def _():
        s_ref[...] = jnp.dot(x_ref[...], w_ref[...],
                             preferred_element_type=jnp.float32)

    bc = adj_ref.shape[1]
    acc = jnp.dot(adj_ref[...], s_ref[pl.ds(0, bc), :],
                  preferred_element_type=jnp.float32)
    acc += jnp.dot(adjt_ref[:, :tail], s_ref[pl.ds(bc, tail), :],
                   preferred_element_type=jnp.float32)
    out_ref[...] = jnp.maximum(acc + b_ref[...], 0.0)


import functools


def kernel(x, adj, W, b):
    N, F = x.shape
    H = W.shape[1]

    BR = 400
    BC = (N // 128) * 128
    tail = N - BC

    out = pl.pallas_call(
        functools.partial(_gcn_kernel, tail),
        grid=(N // BR,),
        in_specs=[
            pl.BlockSpec((N, F), lambda i: (0, 0)),
            pl.BlockSpec((F, H), lambda i: (0, 0)),
            pl.BlockSpec((1, H), lambda i: (0, 0)),
            pl.BlockSpec((BR, BC), lambda i: (i, 0)),
            pl.BlockSpec((BR, 128), lambda i: (i, BC // 128)),
        ],
        out_specs=pl.BlockSpec((BR, H), lambda i: (i, 0)),
        out_shape=jax.ShapeDtypeStruct((N, H), jnp.float32),
        scratch_shapes=[pltpu.VMEM((N, H), jnp.float32)],
        compiler_params=pltpu.CompilerParams(
            dimension_semantics=("arbitrary",),
        ),
    )(x, W, b.reshape(1, H), adj, adj)
    return out
